# baseline (speedup 1.0000x reference)
import functools

import jax
import jax.numpy as jnp
from jax import lax
from jax.experimental import pallas as pl
from jax.experimental.pallas import tpu as pltpu
from jax.experimental.pallas import tpu_sc as plsc

D = 128
E = 320000
NH = 2
EH = E // NH
NV = 10000
NC, NS, L = 2, 16, 16
NWRD = D // 2
WPS = NWRD // NS
E2 = EH // NC
C = 4000
NCH = E2 // C

_mesh = plsc.VectorSubcoreMesh(core_axis_name="c", subcore_axis_name="s")


@functools.partial(
    pl.kernel,
    out_type=jax.ShapeDtypeStruct((NS * EH,), jnp.float32),
    mesh=_mesh,
    scratch_types=[
        pltpu.VMEM((NV,), jnp.int32),
        pltpu.VMEM((NV,), jnp.int32),
        pltpu.VMEM((NV,), jnp.int32),
        pltpu.VMEM((NV,), jnp.int32),
        pltpu.VMEM((NV,), jnp.int32),
        pltpu.VMEM((NV,), jnp.int32),
        pltpu.VMEM((NV,), jnp.int32),
        pltpu.VMEM((NV,), jnp.int32),
        pltpu.VMEM((C,), jnp.int32),
        pltpu.VMEM((C,), jnp.int32),
        pltpu.VMEM((C,), jnp.int32),
        pltpu.VMEM((C,), jnp.int32),
        pltpu.VMEM((C,), jnp.float32),
        pltpu.VMEM((C,), jnp.float32),
        pltpu.SemaphoreType.DMA,
        pltpu.SemaphoreType.DMA,
        pltpu.SemaphoreType.DMA,
        pltpu.SemaphoreType.DMA,
    ],
    compiler_params=pltpu.CompilerParams(needs_layout_passes=False),
)
def _partial_dots(xu_hbm, xi_hbm, src_hbm, dst_hbm, part_hbm,
                  uw0, uw1, uw2, uw3, vw0, vw1, vw2, vw3,
                  sv0, sv1, dv0, dv1, ov0, ov1,
                  qi0, qi1, qo0, qo1):
    cid = lax.axis_index("c")
    sid = lax.axis_index("s")
    ebase = cid * E2

    u_w = (uw0, uw1, uw2, uw3)
    v_w = (vw0, vw1, vw2, vw3)
    for k in range(WPS):
        pltpu.sync_copy(xu_hbm.at[sid * WPS + k], u_w[k])
        pltpu.sync_copy(xi_hbm.at[sid * WPS + k], v_w[k])

    svs, dvs, ovs = (sv0, sv1), (dv0, dv1), (ov0, ov1)
    qis, qos = (qi0, qi1), (qo0, qo1)

    def fire_idx(ci, b):
        off = pl.multiple_of(ebase + ci * C, 8)
        pltpu.async_copy(src_hbm.at[pl.ds(off, C)], svs[b], qis[b])
        pltpu.async_copy(dst_hbm.at[pl.ds(off, C)], dvs[b], qis[b])

    def drain_idx(b):
        pltpu.make_async_copy(src_hbm.at[pl.ds(0, C)], svs[b], qis[b]).wait()
        pltpu.make_async_copy(dst_hbm.at[pl.ds(0, C)], dvs[b], qis[b]).wait()

    def fire_out(ci, b):
        off = pl.multiple_of(sid * EH + ebase + ci * C, 8)
        pltpu.async_copy(ovs[b], part_hbm.at[pl.ds(off, C)], qos[b])

    def drain_out(b):
        pltpu.make_async_copy(
            ovs[b], part_hbm.at[pl.ds(0, C)], qos[b]).wait()

    def compute(ci, b):
        drain_idx(b)

        @plsc.parallel_loop(0, C // L, unroll=8)
        def _grp(g):
            off16 = g * L
            s16 = svs[b][pl.ds(off16, L)]
            d16 = dvs[b][pl.ds(off16, L)]
            acc_e = jnp.zeros((L,), jnp.float32)
            acc_o = jnp.zeros((L,), jnp.float32)
            for k in range(WPS):
                uw = plsc.load_gather(u_w[k], [s16])
                vw = plsc.load_gather(v_w[k], [d16])
                ub = plsc.bitcast(uw, jnp.bfloat16)
                vb = plsc.bitcast(vw, jnp.bfloat16)
                pe, po = plsc.unpack(
                    ub * vb, format=plsc.PackFormat.INTERLEAVED)
                acc_e = acc_e + pe
                acc_o = acc_o + po
            ovs[b][pl.ds(off16, L)] = acc_e + acc_o

    fire_idx(0, 0)

    def step(k, carry):
        i0 = 2 * k
        i1 = i0 + 1
        fire_idx(i1, 1)

        @pl.when(k > 0)
        def _():
            drain_out(0)

        compute(i0, 0)
        fire_out(i0, 0)

        @pl.when(k < NCH // 2 - 1)
        def _():
            fire_idx(i1 + 1, 0)

        @pl.when(k > 0)
        def _():
            drain_out(1)

        compute(i1, 1)
        fire_out(i1, 1)
        return carry

    lax.fori_loop(0, NCH // 2, step, 0)
    drain_out(0)
    drain_out(1)


BK = 32000
NB = EH // BK


def _rowsum_body(p_ref, o_ref):
    o_ref[0, 0, :] = jnp.sum(p_ref[...], axis=0)


_rowsum = pl.pallas_call(
    _rowsum_body,
    out_shape=jax.ShapeDtypeStruct((NB, 1, BK), jnp.float32),
    grid=(NB,),
    in_specs=[pl.BlockSpec((NS, BK), lambda i: (0, i))],
    out_specs=pl.BlockSpec((1, 1, BK), lambda i: (i, 0, 0)),
)


def _pack_words(x):
    u = jax.lax.bitcast_convert_type(x, jnp.uint32)
    b = (u + 0x7FFF + ((u >> 16) & 1)) >> 16
    w = b[:, :NWRD] | (b[:, NWRD:] << 16)
    return w.astype(jnp.int32).T


def kernel(x_user, x_item, edge_label_index):
    eli = edge_label_index.astype(jnp.int32)
    wu, wi = _pack_words(x_user), _pack_words(x_item)
    outs = []
    for h in range(NH):
        sl = slice(h * EH, (h + 1) * EH)
        part = _partial_dots(wu, wi, eli[0, sl], eli[1, sl])
        outs.append(_rowsum(part.reshape(NS, EH)).reshape(EH))
    return jnp.concatenate(outs)


# speedup vs baseline: 1.0571x; 1.0571x over previous
import functools

import jax
import jax.numpy as jnp
from jax import lax
from jax.experimental import pallas as pl
from jax.experimental.pallas import tpu as pltpu
from jax.experimental.pallas import tpu_sc as plsc

D = 128
E = 320000
NV = 10000
NC, NS, L = 2, 16, 16
NWRD = D // 2
WPS = NWRD // NS
E2 = E // NC
C = 4000
NCH = E2 // C
GG = C // (5 * L)

_mesh = plsc.VectorSubcoreMesh(core_axis_name="c", subcore_axis_name="s")


@functools.partial(
    pl.kernel,
    out_type=jax.ShapeDtypeStruct((NS * E,), jnp.float32),
    mesh=_mesh,
    scratch_types=[
        pltpu.VMEM((NV,), jnp.int32),
        pltpu.VMEM((NV,), jnp.int32),
        pltpu.VMEM((NV,), jnp.int32),
        pltpu.VMEM((NV,), jnp.int32),
        pltpu.VMEM((NV,), jnp.int32),
        pltpu.VMEM((NV,), jnp.int32),
        pltpu.VMEM((NV,), jnp.int32),
        pltpu.VMEM((NV,), jnp.int32),
        pltpu.VMEM((C,), jnp.int32),
        pltpu.VMEM((C,), jnp.int32),
        pltpu.VMEM((C,), jnp.int32),
        pltpu.VMEM((C,), jnp.int32),
        pltpu.VMEM((C,), jnp.float32),
        pltpu.VMEM((C,), jnp.float32),
        pltpu.SemaphoreType.DMA,
        pltpu.SemaphoreType.DMA,
        pltpu.SemaphoreType.DMA,
        pltpu.SemaphoreType.DMA,
    ],
    compiler_params=pltpu.CompilerParams(needs_layout_passes=False),
)
def _partial_dots(xu_hbm, xi_hbm, src_hbm, dst_hbm, part_hbm,
                  uw0, uw1, uw2, uw3, vw0, vw1, vw2, vw3,
                  sv0, sv1, dv0, dv1, ov0, ov1,
                  qi0, qi1, qo0, qo1):
    cid = lax.axis_index("c")
    sid = lax.axis_index("s")
    ebase = cid * E2

    u_w = (uw0, uw1, uw2, uw3)
    v_w = (vw0, vw1, vw2, vw3)
    for k in range(WPS):
        pltpu.sync_copy(xu_hbm.at[sid * WPS + k], u_w[k])
        pltpu.sync_copy(xi_hbm.at[sid * WPS + k], v_w[k])

    svs, dvs, ovs = (sv0, sv1), (dv0, dv1), (ov0, ov1)
    qis, qos = (qi0, qi1), (qo0, qo1)

    def fire_idx(ci, b):
        off = pl.multiple_of(ebase + ci * C, 8)
        pltpu.async_copy(src_hbm.at[pl.ds(off, C)], svs[b], qis[b])
        pltpu.async_copy(dst_hbm.at[pl.ds(off, C)], dvs[b], qis[b])

    def drain_idx(b):
        pltpu.make_async_copy(src_hbm.at[pl.ds(0, C)], svs[b], qis[b]).wait()
        pltpu.make_async_copy(dst_hbm.at[pl.ds(0, C)], dvs[b], qis[b]).wait()

    def fire_out(ci, b):
        off = pl.multiple_of(sid * E + ebase + ci * C, 8)
        pltpu.async_copy(ovs[b], part_hbm.at[pl.ds(off, C)], qos[b])

    def drain_out(b):
        pltpu.make_async_copy(
            ovs[b], part_hbm.at[pl.ds(0, C)], qos[b]).wait()

    def compute(ci, b):
        drain_idx(b)

        @plsc.parallel_loop(0, C // L, unroll=8)
        def _grp(g):
            off16 = g * L
            s16 = svs[b][pl.ds(off16, L)]
            d16 = dvs[b][pl.ds(off16, L)]
            acc_e = jnp.zeros((L,), jnp.float32)
            acc_o = jnp.zeros((L,), jnp.float32)
            for k in range(WPS):
                uw = plsc.load_gather(u_w[k], [s16])
                vw = plsc.load_gather(v_w[k], [d16])
                ub = plsc.bitcast(uw, jnp.bfloat16)
                vb = plsc.bitcast(vw, jnp.bfloat16)
                pe, po = plsc.unpack(
                    ub * vb, format=plsc.PackFormat.INTERLEAVED)
                acc_e = acc_e + pe
                acc_o = acc_o + po
            ovs[b][pl.ds(off16, L)] = acc_e + acc_o

    fire_idx(0, 0)

    def step(k, carry):
        i0 = 2 * k
        i1 = i0 + 1
        fire_idx(i1, 1)

        @pl.when(k > 0)
        def _():
            drain_out(0)

        compute(i0, 0)
        fire_out(i0, 0)

        @pl.when(k < NCH // 2 - 1)
        def _():
            fire_idx(i1 + 1, 0)

        @pl.when(k > 0)
        def _():
            drain_out(1)

        compute(i1, 1)
        fire_out(i1, 1)
        return carry

    lax.fori_loop(0, NCH // 2, step, 0)
    drain_out(0)
    drain_out(1)


BK = 32000
NB = E // BK


def _rowsum_body(p_ref, o_ref):
    o_ref[0, 0, :] = jnp.sum(p_ref[...], axis=0)


_rowsum = pl.pallas_call(
    _rowsum_body,
    out_shape=jax.ShapeDtypeStruct((NB, 1, BK), jnp.float32),
    grid=(NB,),
    in_specs=[pl.BlockSpec((NS, BK), lambda i: (0, i))],
    out_specs=pl.BlockSpec((1, 1, BK), lambda i: (i, 0, 0)),
)


def _packT_body(xu_ref, xi_ref, wu_ref, wi_ref):
    for r, w in ((xu_ref, wu_ref), (xi_ref, wi_ref)):
        u = jax.lax.bitcast_convert_type(r[...], jnp.uint32)
        b = (u + 0x7FFF + ((u >> 16) & 1)) >> 16
        word = b[:, :NWRD] | (b[:, NWRD:] << 16)
        w[...] = word.astype(jnp.int32).T


_packT = pl.pallas_call(
    _packT_body,
    out_shape=(jax.ShapeDtypeStruct((NWRD, NV), jnp.int32),
               jax.ShapeDtypeStruct((NWRD, NV), jnp.int32)),
)


def kernel(x_user, x_item, edge_label_index):
    eli = edge_label_index.astype(jnp.int32)
    wu, wi = _packT(x_user, x_item)
    part = _partial_dots(wu, wi, eli[0], eli[1])
    return _rowsum(part.reshape(NS, E)).reshape(E)


# speedup vs baseline: 1.0601x; 1.0028x over previous
import functools

import jax
import jax.numpy as jnp
from jax import lax
from jax.experimental import pallas as pl
from jax.experimental.pallas import tpu as pltpu
from jax.experimental.pallas import tpu_sc as plsc

D = 128
E = 320000
NV = 10000
NC, NS, L = 2, 16, 16
NWRD = D // 2
WPS = NWRD // NS
E2 = E // NC
C = 4000
NCH = E2 // C

_mesh = plsc.VectorSubcoreMesh(core_axis_name="c", subcore_axis_name="s")


@functools.partial(
    pl.kernel,
    out_type=jax.ShapeDtypeStruct((NS * E,), jnp.float32),
    mesh=_mesh,
    scratch_types=[
        pltpu.VMEM((NV,), jnp.int32),
        pltpu.VMEM((NV,), jnp.int32),
        pltpu.VMEM((NV,), jnp.int32),
        pltpu.VMEM((NV,), jnp.int32),
        pltpu.VMEM((NV,), jnp.int32),
        pltpu.VMEM((NV,), jnp.int32),
        pltpu.VMEM((NV,), jnp.int32),
        pltpu.VMEM((NV,), jnp.int32),
        pltpu.VMEM((C,), jnp.int32),
        pltpu.VMEM((C,), jnp.int32),
        pltpu.VMEM((C,), jnp.int32),
        pltpu.VMEM((C,), jnp.int32),
        pltpu.VMEM((C,), jnp.float32),
        pltpu.VMEM((C,), jnp.float32),
        pltpu.SemaphoreType.DMA,
        pltpu.SemaphoreType.DMA,
        pltpu.SemaphoreType.DMA,
        pltpu.SemaphoreType.DMA,
    ],
    compiler_params=pltpu.CompilerParams(needs_layout_passes=False),
)
def _partial_dots(xu_hbm, xi_hbm, src_hbm, dst_hbm, part_hbm,
                  uw0, uw1, uw2, uw3, vw0, vw1, vw2, vw3,
                  sv0, sv1, dv0, dv1, ov0, ov1,
                  qi0, qi1, qo0, qo1):
    cid = lax.axis_index("c")
    sid = lax.axis_index("s")
    ebase = cid * E2

    u_w = (uw0, uw1, uw2, uw3)
    v_w = (vw0, vw1, vw2, vw3)
    for k in range(WPS):
        pltpu.sync_copy(xu_hbm.at[sid * WPS + k], u_w[k])
        pltpu.sync_copy(xi_hbm.at[sid * WPS + k], v_w[k])

    svs, dvs, ovs = (sv0, sv1), (dv0, dv1), (ov0, ov1)
    qis, qos = (qi0, qi1), (qo0, qo1)

    def fire_idx(ci, b):
        off = pl.multiple_of(ebase + ci * C, 8)
        pltpu.async_copy(src_hbm.at[pl.ds(off, C)], svs[b], qis[b])
        pltpu.async_copy(dst_hbm.at[pl.ds(off, C)], dvs[b], qis[b])

    def drain_idx(b):
        pltpu.make_async_copy(src_hbm.at[pl.ds(0, C)], svs[b], qis[b]).wait()
        pltpu.make_async_copy(dst_hbm.at[pl.ds(0, C)], dvs[b], qis[b]).wait()

    def fire_out(ci, b):
        off = pl.multiple_of(sid * E + ebase + ci * C, 8)
        pltpu.async_copy(ovs[b], part_hbm.at[pl.ds(off, C)], qos[b])

    def drain_out(b):
        pltpu.make_async_copy(
            ovs[b], part_hbm.at[pl.ds(0, C)], qos[b]).wait()

    def compute(ci, b):
        drain_idx(b)

        @plsc.parallel_loop(0, C // L, unroll=8)
        def _grp(g):
            off16 = g * L
            s16 = svs[b][pl.ds(off16, L)]
            d16 = dvs[b][pl.ds(off16, L)]
            acc_e = jnp.zeros((L,), jnp.float32)
            acc_o = jnp.zeros((L,), jnp.float32)
            for k in range(WPS):
                uw = plsc.load_gather(u_w[k], [s16])
                vw = plsc.load_gather(v_w[k], [d16])
                ub = plsc.bitcast(uw, jnp.bfloat16)
                vb = plsc.bitcast(vw, jnp.bfloat16)
                pe, po = plsc.unpack(
                    ub * vb, format=plsc.PackFormat.INTERLEAVED)
                acc_e = acc_e + pe
                acc_o = acc_o + po
            ovs[b][pl.ds(off16, L)] = acc_e + acc_o

    fire_idx(0, 0)

    def step(k, carry):
        i0 = 2 * k
        i1 = i0 + 1
        fire_idx(i1, 1)

        @pl.when(k > 0)
        def _():
            drain_out(0)

        compute(i0, 0)
        fire_out(i0, 0)

        @pl.when(k < NCH // 2 - 1)
        def _():
            fire_idx(i1 + 1, 0)

        @pl.when(k > 0)
        def _():
            drain_out(1)

        compute(i1, 1)
        fire_out(i1, 1)
        return carry

    lax.fori_loop(0, NCH // 2, step, 0)
    drain_out(0)
    drain_out(1)


BK = 32000
NB = E // BK


def _rowsum_body(p_ref, o_ref):
    o_ref[0, 0, :] = jnp.sum(p_ref[...], axis=0)


_rowsum = pl.pallas_call(
    _rowsum_body,
    out_shape=jax.ShapeDtypeStruct((NB, 1, BK), jnp.float32),
    grid=(NB,),
    in_specs=[pl.BlockSpec((NS, BK), lambda i: (0, i))],
    out_specs=pl.BlockSpec((1, 1, BK), lambda i: (i, 0, 0)),
)


def _packT_body(xu_ref, xi_ref, wu_ref, wi_ref):
    for r, w in ((xu_ref, wu_ref), (xi_ref, wi_ref)):
        u = jax.lax.bitcast_convert_type(r[...], jnp.uint32)
        b = (u + 0x7FFF + ((u >> 16) & 1)) >> 16
        word = b[:, :NWRD] | (b[:, NWRD:] << 16)
        w[...] = word.astype(jnp.int32).T


_packT = pl.pallas_call(
    _packT_body,
    out_shape=(jax.ShapeDtypeStruct((NWRD, NV), jnp.int32),
               jax.ShapeDtypeStruct((NWRD, NV), jnp.int32)),
)


def kernel(x_user, x_item, edge_label_index):
    eli = edge_label_index.astype(jnp.int32)
    wu, wi = _packT(x_user, x_item)
    part = _partial_dots(wu, wi, eli[0], eli[1])
    return _rowsum(part.reshape(NS, E)).reshape(E)
